# Initial kernel scaffold; baseline (speedup 1.0000x reference)
import jax
import jax.numpy as jnp
from jax.experimental import pallas as pl


def kernel(inputs, classes, embeddings_weight):
    raise NotImplementedError("write your pallas kernel here")


# speedup vs baseline: 1.8120x; 1.8120x over previous
import jax
import jax.numpy as jnp
from jax.experimental import pallas as pl
from jax.experimental.pallas import tpu as pltpu

_B = 16384
_K = 1024
_D = 256
_R = 256
_COMMITMENT_COST = 0.25


def _vq_block_kernel(x_ref, sx_ref, se_ref, w_ref,
                     enc_ref, q_ref, loss_ref, perp_ref,
                     cnt_ref, acc_ref):
    i = pl.program_id(0)

    @pl.when(i == 0)
    def _init():
        cnt_ref[...] = jnp.zeros_like(cnt_ref)
        acc_ref[...] = jnp.zeros_like(acc_ref)

    x = x_ref[...]
    w = w_ref[...]
    m = jax.lax.dot_general(x, w, (((1,), (1,)), ((), ())),
                            preferred_element_type=jnp.float32)
    d = (sx_ref[...] + se_ref[...]) - 2.0 * m
    dmin = jnp.min(d, axis=1, keepdims=True)
    cols = jax.lax.broadcasted_iota(jnp.int32, (d.shape[0], d.shape[1]), 1)
    idx = jnp.min(jnp.where(d == dmin, cols, _K), axis=1, keepdims=True)
    onehot = (cols == idx).astype(jnp.float32)
    enc_ref[...] = onehot
    q = jax.lax.dot_general(onehot, w, (((1,), (0,)), ((), ())),
                            preferred_element_type=jnp.float32)
    q_ref[...] = x + (q - x)
    cnt_ref[...] += jnp.sum(onehot, axis=0, keepdims=True)
    acc_ref[...] += jnp.sum(dmin).reshape(1, 1)

    @pl.when(i == pl.num_programs(0) - 1)
    def _fin():
        total = acc_ref[...]
        loss_ref[...] = (1.0 + _COMMITMENT_COST) * total / (_B * _D)
        p = cnt_ref[...] / _B
        perp_ref[...] = jnp.exp(-jnp.sum(p * jnp.log(p + 1e-10))).reshape(1, 1)


def kernel(inputs, classes, embeddings_weight):
    del classes
    input_shape = inputs.shape
    x = inputs.reshape(_B, _D)
    sx = jnp.sum(x ** 2, axis=1, keepdims=True)
    se = jnp.sum(embeddings_weight ** 2, axis=1)[None, :]
    grid = _B // _R
    enc, q, loss, perp = pl.pallas_call(
        _vq_block_kernel,
        grid=(grid,),
        in_specs=[
            pl.BlockSpec((_R, _D), lambda i: (i, 0)),
            pl.BlockSpec((_R, 1), lambda i: (i, 0)),
            pl.BlockSpec((1, _K), lambda i: (0, 0)),
            pl.BlockSpec((_K, _D), lambda i: (0, 0)),
        ],
        out_specs=[
            pl.BlockSpec((_R, _K), lambda i: (i, 0)),
            pl.BlockSpec((_R, _D), lambda i: (i, 0)),
            pl.BlockSpec((1, 1), lambda i: (0, 0)),
            pl.BlockSpec((1, 1), lambda i: (0, 0)),
        ],
        out_shape=[
            jax.ShapeDtypeStruct((_B, _K), jnp.float32),
            jax.ShapeDtypeStruct((_B, _D), jnp.float32),
            jax.ShapeDtypeStruct((1, 1), jnp.float32),
            jax.ShapeDtypeStruct((1, 1), jnp.float32),
        ],
        scratch_shapes=[
            pltpu.VMEM((1, _K), jnp.float32),
            pltpu.VMEM((1, 1), jnp.float32),
        ],
    )(x, sx, se, embeddings_weight)
    return (loss[0, 0], q.reshape(input_shape), perp[0, 0], enc)


# speedup vs baseline: 2.0805x; 1.1482x over previous
import jax
import jax.numpy as jnp
from jax.experimental import pallas as pl
from jax.experimental.pallas import tpu as pltpu

_B = 16384
_K = 1024
_D = 256
_R = 512
_COMMITMENT_COST = 0.25


def _vq_block_kernel(x_ref, sx_ref, se_ref, w_ref,
                     enc_ref, q_ref, loss_ref, perp_ref,
                     cnt_ref, acc_ref):
    i = pl.program_id(0)

    @pl.when(i == 0)
    def _init():
        cnt_ref[...] = jnp.zeros_like(cnt_ref)
        acc_ref[...] = jnp.zeros_like(acc_ref)

    x = x_ref[...]
    w = w_ref[...]
    m = jax.lax.dot_general(x, w, (((1,), (1,)), ((), ())),
                            preferred_element_type=jnp.float32)
    d = (sx_ref[...] + se_ref[...]) - 2.0 * m
    dmin = jnp.min(d, axis=1, keepdims=True)
    cols = jax.lax.broadcasted_iota(jnp.int32, (d.shape[0], d.shape[1]), 1)
    idx = jnp.min(jnp.where(d == dmin, cols, _K), axis=1, keepdims=True)
    onehot = (cols == idx).astype(jnp.float32)
    enc_ref[...] = onehot
    q = jax.lax.dot_general(onehot, w, (((1,), (0,)), ((), ())),
                            preferred_element_type=jnp.float32)
    q_ref[...] = x + (q - x)
    cnt_ref[...] += jnp.sum(onehot, axis=0, keepdims=True)
    acc_ref[...] += jnp.sum(dmin).reshape(1, 1)

    @pl.when(i == pl.num_programs(0) - 1)
    def _fin():
        total = acc_ref[...]
        loss_ref[...] = (1.0 + _COMMITMENT_COST) * total / (_B * _D)
        p = cnt_ref[...] / _B
        perp_ref[...] = jnp.exp(-jnp.sum(p * jnp.log(p + 1e-10))).reshape(1, 1)


def kernel(inputs, classes, embeddings_weight):
    del classes
    input_shape = inputs.shape
    x = inputs.reshape(_B, _D)
    sx = jnp.sum(x ** 2, axis=1, keepdims=True)
    se = jnp.sum(embeddings_weight ** 2, axis=1)[None, :]
    grid = _B // _R
    enc, q, loss, perp = pl.pallas_call(
        _vq_block_kernel,
        grid=(grid,),
        in_specs=[
            pl.BlockSpec((_R, _D), lambda i: (i, 0)),
            pl.BlockSpec((_R, 1), lambda i: (i, 0)),
            pl.BlockSpec((1, _K), lambda i: (0, 0)),
            pl.BlockSpec((_K, _D), lambda i: (0, 0)),
        ],
        out_specs=[
            pl.BlockSpec((_R, _K), lambda i: (i, 0)),
            pl.BlockSpec((_R, _D), lambda i: (i, 0)),
            pl.BlockSpec((1, 1), lambda i: (0, 0)),
            pl.BlockSpec((1, 1), lambda i: (0, 0)),
        ],
        out_shape=[
            jax.ShapeDtypeStruct((_B, _K), jnp.float32),
            jax.ShapeDtypeStruct((_B, _D), jnp.float32),
            jax.ShapeDtypeStruct((1, 1), jnp.float32),
            jax.ShapeDtypeStruct((1, 1), jnp.float32),
        ],
        scratch_shapes=[
            pltpu.VMEM((1, _K), jnp.float32),
            pltpu.VMEM((1, 1), jnp.float32),
        ],
    )(x, sx, se, embeddings_weight)
    return (loss[0, 0], q.reshape(input_shape), perp[0, 0], enc)


# speedup vs baseline: 2.1730x; 1.0444x over previous
import jax
import jax.numpy as jnp
from jax.experimental import pallas as pl
from jax.experimental.pallas import tpu as pltpu

_B = 16384
_K = 1024
_D = 256
_R = 1024
_COMMITMENT_COST = 0.25


def _vq_block_kernel(x_ref, sx_ref, se_ref, w_ref,
                     enc_ref, q_ref, loss_ref, perp_ref,
                     cnt_ref, acc_ref):
    i = pl.program_id(0)

    @pl.when(i == 0)
    def _init():
        cnt_ref[...] = jnp.zeros_like(cnt_ref)
        acc_ref[...] = jnp.zeros_like(acc_ref)

    x = x_ref[...]
    w = w_ref[...]
    m = jax.lax.dot_general(x, w, (((1,), (1,)), ((), ())),
                            preferred_element_type=jnp.float32)
    d = (sx_ref[...] + se_ref[...]) - 2.0 * m
    dmin = jnp.min(d, axis=1, keepdims=True)
    cols = jax.lax.broadcasted_iota(jnp.int32, (d.shape[0], d.shape[1]), 1)
    idx = jnp.min(jnp.where(d == dmin, cols, _K), axis=1, keepdims=True)
    onehot = (cols == idx).astype(jnp.float32)
    enc_ref[...] = onehot
    q = jax.lax.dot_general(onehot, w, (((1,), (0,)), ((), ())),
                            preferred_element_type=jnp.float32)
    q_ref[...] = x + (q - x)
    cnt_ref[...] += jnp.sum(onehot, axis=0, keepdims=True)
    acc_ref[...] += jnp.sum(dmin).reshape(1, 1)

    @pl.when(i == pl.num_programs(0) - 1)
    def _fin():
        total = acc_ref[...]
        loss_ref[...] = (1.0 + _COMMITMENT_COST) * total / (_B * _D)
        p = cnt_ref[...] / _B
        perp_ref[...] = jnp.exp(-jnp.sum(p * jnp.log(p + 1e-10))).reshape(1, 1)


def kernel(inputs, classes, embeddings_weight):
    del classes
    input_shape = inputs.shape
    x = inputs.reshape(_B, _D)
    sx = jnp.sum(x ** 2, axis=1, keepdims=True)
    se = jnp.sum(embeddings_weight ** 2, axis=1)[None, :]
    grid = _B // _R
    enc, q, loss, perp = pl.pallas_call(
        _vq_block_kernel,
        grid=(grid,),
        in_specs=[
            pl.BlockSpec((_R, _D), lambda i: (i, 0)),
            pl.BlockSpec((_R, 1), lambda i: (i, 0)),
            pl.BlockSpec((1, _K), lambda i: (0, 0)),
            pl.BlockSpec((_K, _D), lambda i: (0, 0)),
        ],
        out_specs=[
            pl.BlockSpec((_R, _K), lambda i: (i, 0)),
            pl.BlockSpec((_R, _D), lambda i: (i, 0)),
            pl.BlockSpec((1, 1), lambda i: (0, 0)),
            pl.BlockSpec((1, 1), lambda i: (0, 0)),
        ],
        out_shape=[
            jax.ShapeDtypeStruct((_B, _K), jnp.float32),
            jax.ShapeDtypeStruct((_B, _D), jnp.float32),
            jax.ShapeDtypeStruct((1, 1), jnp.float32),
            jax.ShapeDtypeStruct((1, 1), jnp.float32),
        ],
        scratch_shapes=[
            pltpu.VMEM((1, _K), jnp.float32),
            pltpu.VMEM((1, 1), jnp.float32),
        ],
    )(x, sx, se, embeddings_weight)
    return (loss[0, 0], q.reshape(input_shape), perp[0, 0], enc)


# speedup vs baseline: 2.2221x; 1.0226x over previous
import jax
import jax.numpy as jnp
from jax.experimental import pallas as pl
from jax.experimental.pallas import tpu as pltpu

_B = 16384
_K = 1024
_D = 256
_R = 2048
_COMMITMENT_COST = 0.25


def _vq_block_kernel(x_ref, sx_ref, se_ref, w_ref,
                     enc_ref, q_ref, loss_ref, perp_ref,
                     cnt_ref, acc_ref):
    i = pl.program_id(0)

    @pl.when(i == 0)
    def _init():
        cnt_ref[...] = jnp.zeros_like(cnt_ref)
        acc_ref[...] = jnp.zeros_like(acc_ref)

    x = x_ref[...]
    w = w_ref[...]
    m = jax.lax.dot_general(x, w, (((1,), (1,)), ((), ())),
                            preferred_element_type=jnp.float32)
    d = (sx_ref[...] + se_ref[...]) - 2.0 * m
    dmin = jnp.min(d, axis=1, keepdims=True)
    cols = jax.lax.broadcasted_iota(jnp.int32, (d.shape[0], d.shape[1]), 1)
    idx = jnp.min(jnp.where(d == dmin, cols, _K), axis=1, keepdims=True)
    onehot = (cols == idx).astype(jnp.float32)
    enc_ref[...] = onehot
    q = jax.lax.dot_general(onehot, w, (((1,), (0,)), ((), ())),
                            preferred_element_type=jnp.float32)
    q_ref[...] = x + (q - x)
    cnt_ref[...] += jnp.sum(onehot, axis=0, keepdims=True)
    acc_ref[...] += jnp.sum(dmin).reshape(1, 1)

    @pl.when(i == pl.num_programs(0) - 1)
    def _fin():
        total = acc_ref[...]
        loss_ref[...] = (1.0 + _COMMITMENT_COST) * total / (_B * _D)
        p = cnt_ref[...] / _B
        perp_ref[...] = jnp.exp(-jnp.sum(p * jnp.log(p + 1e-10))).reshape(1, 1)


def kernel(inputs, classes, embeddings_weight):
    del classes
    input_shape = inputs.shape
    x = inputs.reshape(_B, _D)
    sx = jnp.sum(x ** 2, axis=1, keepdims=True)
    se = jnp.sum(embeddings_weight ** 2, axis=1)[None, :]
    grid = _B // _R
    enc, q, loss, perp = pl.pallas_call(
        _vq_block_kernel,
        grid=(grid,),
        in_specs=[
            pl.BlockSpec((_R, _D), lambda i: (i, 0)),
            pl.BlockSpec((_R, 1), lambda i: (i, 0)),
            pl.BlockSpec((1, _K), lambda i: (0, 0)),
            pl.BlockSpec((_K, _D), lambda i: (0, 0)),
        ],
        out_specs=[
            pl.BlockSpec((_R, _K), lambda i: (i, 0)),
            pl.BlockSpec((_R, _D), lambda i: (i, 0)),
            pl.BlockSpec((1, 1), lambda i: (0, 0)),
            pl.BlockSpec((1, 1), lambda i: (0, 0)),
        ],
        out_shape=[
            jax.ShapeDtypeStruct((_B, _K), jnp.float32),
            jax.ShapeDtypeStruct((_B, _D), jnp.float32),
            jax.ShapeDtypeStruct((1, 1), jnp.float32),
            jax.ShapeDtypeStruct((1, 1), jnp.float32),
        ],
        scratch_shapes=[
            pltpu.VMEM((1, _K), jnp.float32),
            pltpu.VMEM((1, 1), jnp.float32),
        ],
    )(x, sx, se, embeddings_weight)
    return (loss[0, 0], q.reshape(input_shape), perp[0, 0], enc)
